# baseline (speedup 1.0000x reference)
import jax
import jax.numpy as jnp
from jax.experimental import pallas as pl
from jax.experimental.pallas import tpu as pltpu

_BS = 2048
_NBUF = 2


def _pipeline_body(w_hbm, out_hbm, buf, in_sems, out_sems):
    nb = out_hbm.shape[0]
    n = w_hbm.shape[0] // _BS
    pending = [None] * _NBUF

    def start_in(i, slot):
        cp = pltpu.make_async_copy(
            w_hbm.at[pl.ds(i * _BS, _BS)], buf.at[slot], in_sems.at[slot])
        cp.start()
        return cp

    def start_writes(i, slot):
        cps = [
            pltpu.make_async_copy(
                buf.at[slot], out_hbm.at[b, pl.ds(i * _BS, _BS)],
                out_sems.at[slot])
            for b in range(nb)
        ]
        for c in cps:
            c.start()
        return cps

    def drain(cps):
        if cps:
            for c in cps:
                c.wait()

    in_flight = [None] * _NBUF
    in_flight[0] = start_in(0, 0)
    for i in range(n):
        slot = i % _NBUF
        if i + 1 < n:
            nslot = (i + 1) % _NBUF
            drain(pending[nslot])
            pending[nslot] = None
            in_flight[nslot] = start_in(i + 1, nslot)
        in_flight[slot].wait()
        pending[slot] = start_writes(i, slot)
    for cps in pending:
        drain(cps)


def kernel(x, W):
    B, S, H = x.shape
    return pl.pallas_call(
        _pipeline_body,
        in_specs=[pl.BlockSpec(memory_space=pl.ANY)],
        out_specs=pl.BlockSpec(memory_space=pl.ANY),
        out_shape=jax.ShapeDtypeStruct((B, S, H), W.dtype),
        scratch_shapes=[
            pltpu.VMEM((_NBUF, _BS, H), W.dtype),
            pltpu.SemaphoreType.DMA((_NBUF,)),
            pltpu.SemaphoreType.DMA((_NBUF,)),
        ],
    )(W[:S])


# speedup vs baseline: 1.0184x; 1.0184x over previous
import jax
import jax.numpy as jnp
from jax.experimental import pallas as pl
from jax.experimental.pallas import tpu as pltpu

_BS = 1024
_NBUF = 4


def _pipeline_body(w_hbm, out_hbm, buf, in_sems, out_sems):
    nb = out_hbm.shape[0]
    n = w_hbm.shape[0] // _BS
    pending = [None] * _NBUF

    def start_in(i, slot):
        cp = pltpu.make_async_copy(
            w_hbm.at[pl.ds(i * _BS, _BS)], buf.at[slot], in_sems.at[slot])
        cp.start()
        return cp

    def start_writes(i, slot):
        cps = [
            pltpu.make_async_copy(
                buf.at[slot], out_hbm.at[b, pl.ds(i * _BS, _BS)],
                out_sems.at[slot])
            for b in range(nb)
        ]
        for c in cps:
            c.start()
        return cps

    def drain(cps):
        if cps:
            for c in cps:
                c.wait()

    in_flight = [None] * _NBUF
    in_flight[0] = start_in(0, 0)
    for i in range(n):
        slot = i % _NBUF
        if i + 1 < n:
            nslot = (i + 1) % _NBUF
            drain(pending[nslot])
            pending[nslot] = None
            in_flight[nslot] = start_in(i + 1, nslot)
        in_flight[slot].wait()
        pending[slot] = start_writes(i, slot)
    for cps in pending:
        drain(cps)


def kernel(x, W):
    B, S, H = x.shape
    return pl.pallas_call(
        _pipeline_body,
        in_specs=[pl.BlockSpec(memory_space=pl.ANY)],
        out_specs=pl.BlockSpec(memory_space=pl.ANY),
        out_shape=jax.ShapeDtypeStruct((B, S, H), W.dtype),
        scratch_shapes=[
            pltpu.VMEM((_NBUF, _BS, H), W.dtype),
            pltpu.SemaphoreType.DMA((_NBUF,)),
            pltpu.SemaphoreType.DMA((_NBUF,)),
        ],
    )(W[:S])
